# Initial kernel scaffold; baseline (speedup 1.0000x reference)
import jax
import jax.numpy as jnp
from jax.experimental import pallas as pl


def kernel(x, edge_index, edge_attr, W1, b1, W2, b2, W3, b3, Wc, bc):
    raise NotImplementedError("write your pallas kernel here")


# speedup vs baseline: 5.4381x; 5.4381x over previous
import functools

import jax
import jax.numpy as jnp
from jax import lax
from jax.experimental import pallas as pl
from jax.experimental.pallas import tpu as pltpu
from jax.experimental.pallas import tpu_sc as plsc

N_NODES = 10000
N_EDGES = 160000
NC = 2
NS = 16
LANES = 16
CHUNK = 128

N_CHUNK_MAIN = 80
E_PAD = NS * N_CHUNK_MAIN * CHUNK
N_CHUNK_DEG = E_PAD // (NC * NS * CHUNK)

N_FULL_SLABS = N_NODES // CHUNK
TAIL_ROWS = N_NODES - N_FULL_SLABS * CHUNK
SLABS_PER_TILE = 5

RB = 1000
GRID = N_NODES // RB


def _make_agg(feat, split_cores, gather=True):
    n_chunk = N_CHUNK_DEG if split_cores else N_CHUNK_MAIN
    mesh = plsc.VectorSubcoreMesh(
        core_axis_name="c", subcore_axis_name="s", num_cores=NC, num_subcores=NS
    )

    def body(*refs):
        if gather:
            hs, src, dst, w, out, acc, srcbuf, dstbuf, wbuf, rows, sem = refs
        else:
            dst, w, out, acc, dstbuf, wbuf, rows, sem = refs
            hs = src = srcbuf = None
        c = lax.axis_index("c")
        s = lax.axis_index("s")

        zeros16 = jnp.zeros((LANES,), jnp.float32)

        def zrow(r, carry):
            for q in range(feat // LANES):
                rows[r, pl.ds(q * LANES, LANES)] = zeros16
            return carry

        lax.fori_loop(0, CHUNK, zrow, 0)
        for m in range(SLABS_PER_TILE):
            k = s + NS * m
            off = pl.multiple_of(k * CHUNK, CHUNK)

            @pl.when(k < N_FULL_SLABS)
            def _():
                pltpu.sync_copy(rows.at[pl.ds(0, CHUNK)], acc.at[pl.ds(off, CHUNK)])

            @pl.when(k == N_FULL_SLABS)
            def _():
                pltpu.sync_copy(rows.at[pl.ds(0, TAIL_ROWS)],
                                acc.at[pl.ds(off, TAIL_ROWS)])

        plsc.subcore_barrier()

        tile = c * NS + s if split_cores else s
        base_off = c * N_NODES

        def chunk_body(i, carry):
            e0 = pl.multiple_of((tile * n_chunk + i) * CHUNK, CHUNK)
            pltpu.sync_copy(dst.at[pl.ds(e0, CHUNK)], dstbuf)
            pltpu.sync_copy(w.at[pl.ds(e0, CHUNK)], wbuf)
            if gather:
                pltpu.sync_copy(src.at[pl.ds(e0, CHUNK)], srcbuf)
                if not split_cores:
                    for q in range(CHUNK // LANES):
                        sl = pl.ds(q * LANES, LANES)
                        srcbuf[sl] = srcbuf[sl] + base_off
                pltpu.async_copy(hs.at[srcbuf], rows, sem).wait()

            def scale(g, carry2):
                wv = wbuf[pl.ds(g * LANES, LANES)]
                for l in range(LANES):
                    wj = wv[l]
                    j = g * LANES + l
                    for q in range(feat // LANES):
                        sl = pl.ds(q * LANES, LANES)
                        if gather:
                            rows[j, sl] = rows[j, sl] * wj
                        else:
                            rows[j, sl] = rows[j, sl] * 0.0 + wj
                return carry2

            lax.fori_loop(0, CHUNK // LANES, scale, 0)
            pltpu.sync_copy(rows, acc.at[dstbuf], add=True)
            return carry

        lax.fori_loop(0, n_chunk, chunk_body, 0)
        plsc.subcore_barrier()

        for m in range(SLABS_PER_TILE):
            k = s + NS * m
            off = pl.multiple_of(k * CHUNK, CHUNK)

            @pl.when(k < N_FULL_SLABS)
            def _():
                pltpu.sync_copy(acc.at[pl.ds(off, CHUNK)],
                                out.at[pl.ds(base_off + off, CHUNK)])

            @pl.when(k == N_FULL_SLABS)
            def _():
                pltpu.sync_copy(acc.at[pl.ds(off, TAIL_ROWS)],
                                out.at[pl.ds(base_off + off, TAIL_ROWS)])

    scratch = [pltpu.VMEM_SHARED((N_NODES, feat), jnp.float32)]
    if gather:
        scratch.append(pltpu.VMEM((CHUNK,), jnp.int32))
    scratch += [
        pltpu.VMEM((CHUNK,), jnp.int32),
        pltpu.VMEM((CHUNK,), jnp.float32),
        pltpu.VMEM((CHUNK, feat), jnp.float32),
        pltpu.SemaphoreType.DMA,
    ]
    return pl.kernel(
        body,
        out_type=jax.ShapeDtypeStruct((NC * N_NODES, feat), jnp.float32),
        mesh=mesh,
        scratch_types=scratch,
    )


_agg_deg = _make_agg(128, split_cores=True, gather=False)
_agg_128 = _make_agg(128, split_cores=False)
_agg_64p = _make_agg(128, split_cores=True)


def _leaky(v):
    return jnp.where(v >= 0, v, 0.01 * v)


def _stage_a_body(degp_ref, x_ref, xs_ref, dinv_ref):
    dp = degp_ref[...]
    deg = dp[:, 0:1] + dp[:, 1:2] + 1.0
    dinv = jnp.where(deg > 0, lax.rsqrt(deg), 0.0)
    dinv_ref[...] = dinv
    xp = x_ref[...] * dinv
    xs_ref[0] = xp[:, :128]
    xs_ref[1] = xp[:, 128:]


_stage_a = pl.pallas_call(
    _stage_a_body,
    grid=(GRID,),
    in_specs=[
        pl.BlockSpec((RB, 2), lambda i: (i, 0)),
        pl.BlockSpec((RB, 256), lambda i: (i, 0)),
    ],
    out_specs=[
        pl.BlockSpec((2, RB, 128), lambda i: (0, i, 0)),
        pl.BlockSpec((RB, 1), lambda i: (i, 0)),
    ],
    out_shape=[
        jax.ShapeDtypeStruct((2, N_NODES, 128), jnp.float32),
        jax.ShapeDtypeStruct((N_NODES, 1), jnp.float32),
    ],
)


def _stage_c_body(s1_ref, xs_ref, dinv_ref, w1_ref, b1_ref, w2_ref, o_ref):
    d = dinv_ref[...]
    u0 = (s1_ref[0] + xs_ref[0]) * d
    u1 = (s1_ref[1] + xs_ref[1]) * d
    h = jnp.dot(u0, w1_ref[:128, :], preferred_element_type=jnp.float32)
    h = h + jnp.dot(u1, w1_ref[128:, :], preferred_element_type=jnp.float32)
    h = _leaky(h + b1_ref[...])
    t2 = jnp.dot(h, w2_ref[...], preferred_element_type=jnp.float32) * d
    o_ref[0] = t2[:, :128]
    o_ref[1] = t2[:, 128:]


_stage_c = pl.pallas_call(
    _stage_c_body,
    grid=(GRID,),
    in_specs=[
        pl.BlockSpec((2, RB, 128), lambda i: (0, i, 0)),
        pl.BlockSpec((2, RB, 128), lambda i: (0, i, 0)),
        pl.BlockSpec((RB, 1), lambda i: (i, 0)),
        pl.BlockSpec((256, 512), lambda i: (0, 0)),
        pl.BlockSpec((1, 512), lambda i: (0, 0)),
        pl.BlockSpec((512, 256), lambda i: (0, 0)),
    ],
    out_specs=pl.BlockSpec((2, RB, 128), lambda i: (0, i, 0)),
    out_shape=jax.ShapeDtypeStruct((2, N_NODES, 128), jnp.float32),
)


def _stage_e_body(s2_ref, t2s_ref, dinv_ref, b2_ref, w3_ref, o_ref):
    d = dinv_ref[...]
    u0 = (s2_ref[0] + t2s_ref[0]) * d
    u1 = (s2_ref[1] + t2s_ref[1]) * d
    h2a = _leaky(u0 + b2_ref[:, :128])
    h2b = _leaky(u1 + b2_ref[:, 128:])
    t3 = jnp.dot(h2a, w3_ref[:128, :], preferred_element_type=jnp.float32)
    t3 = t3 + jnp.dot(h2b, w3_ref[128:, :], preferred_element_type=jnp.float32)
    t3 = t3 * d
    o_ref[...] = jnp.concatenate([t3, jnp.zeros((RB, 64), jnp.float32)], axis=1)


_stage_e = pl.pallas_call(
    _stage_e_body,
    grid=(GRID,),
    in_specs=[
        pl.BlockSpec((2, RB, 128), lambda i: (0, i, 0)),
        pl.BlockSpec((2, RB, 128), lambda i: (0, i, 0)),
        pl.BlockSpec((RB, 1), lambda i: (i, 0)),
        pl.BlockSpec((1, 256), lambda i: (0, 0)),
        pl.BlockSpec((256, 64), lambda i: (0, 0)),
    ],
    out_specs=pl.BlockSpec((RB, 128), lambda i: (i, 0)),
    out_shape=jax.ShapeDtypeStruct((N_NODES, 128), jnp.float32),
)


def _stage_g_body(s3_ref, t3s_ref, dinv_ref, b3_ref, wc_ref, bc_ref, o_ref):
    d = dinv_ref[...]
    u = (s3_ref[0, :, :64] + s3_ref[1, :, :64] + t3s_ref[:, :64]) * d
    h3 = _leaky(u + b3_ref[...])
    o = jnp.dot(h3, wc_ref[...], preferred_element_type=jnp.float32)
    o_ref[...] = o + bc_ref[...]


_stage_g = pl.pallas_call(
    _stage_g_body,
    grid=(GRID,),
    in_specs=[
        pl.BlockSpec((2, RB, 128), lambda i: (0, i, 0)),
        pl.BlockSpec((RB, 128), lambda i: (i, 0)),
        pl.BlockSpec((RB, 1), lambda i: (i, 0)),
        pl.BlockSpec((1, 64), lambda i: (0, 0)),
        pl.BlockSpec((64, 128), lambda i: (0, 0)),
        pl.BlockSpec((1, 128), lambda i: (0, 0)),
    ],
    out_specs=pl.BlockSpec((RB, 128), lambda i: (i, 0)),
    out_shape=jax.ShapeDtypeStruct((N_NODES, 128), jnp.float32),
)


def kernel(x, edge_index, edge_attr, W1, b1, W2, b2, W3, b3, Wc, bc):
    src = edge_index[0].astype(jnp.int32)
    dst = edge_index[1].astype(jnp.int32)
    w = edge_attr.astype(jnp.float32)
    pad = E_PAD - N_EDGES
    src = jnp.concatenate([src, jnp.zeros((pad,), jnp.int32)])
    dst = jnp.concatenate([dst, jnp.zeros((pad,), jnp.int32)])
    w = jnp.concatenate([w, jnp.zeros((pad,), jnp.float32)])

    degp = _agg_deg(dst, w)
    degp2 = jnp.stack([degp[:N_NODES, 0], degp[N_NODES:, 0]], axis=1)

    xs, dinv = _stage_a(degp2, x)
    s1 = _agg_128(xs.reshape(NC * N_NODES, 128), src, dst, w)
    t2s = _stage_c(s1.reshape(2, N_NODES, 128), xs, dinv, W1,
                   b1.reshape(1, 512), W2)
    s2 = _agg_128(t2s.reshape(NC * N_NODES, 128), src, dst, w)
    t3s = _stage_e(s2.reshape(2, N_NODES, 128), t2s, dinv,
                   b2.reshape(1, 256), W3)
    s3 = _agg_64p(t3s, src, dst, w)
    wcp = jnp.pad(Wc, ((0, 0), (0, 125)))
    bcp = jnp.pad(bc, (0, 125)).reshape(1, 128)
    outp = _stage_g(s3.reshape(2, N_NODES, 128), t3s, dinv,
                    b3.reshape(1, 64), wcp, bcp)
    return outp[:, :3]


# speedup vs baseline: 8.9212x; 1.6405x over previous
import functools

import jax
import jax.numpy as jnp
from jax import lax
from jax.experimental import pallas as pl
from jax.experimental.pallas import tpu as pltpu
from jax.experimental.pallas import tpu_sc as plsc

N_NODES = 10000
N_EDGES = 160000
NC = 2
NS = 16
LANES = 16
CHUNK = 128

N_CHUNK_MAIN = 80
E_PAD = NS * N_CHUNK_MAIN * CHUNK
N_CHUNK_DEG = E_PAD // (NC * NS * CHUNK)

N_FULL_SLABS = N_NODES // CHUNK
TAIL_ROWS = N_NODES - N_FULL_SLABS * CHUNK
SLABS_PER_TILE = 5

RB = 1000
GRID = N_NODES // RB


def _make_agg(feat, split_cores, gather=True):
    n_chunk = N_CHUNK_DEG if split_cores else N_CHUNK_MAIN
    mesh = plsc.VectorSubcoreMesh(
        core_axis_name="c", subcore_axis_name="s", num_cores=NC, num_subcores=NS
    )

    def body(*refs):
        if gather:
            (hs, src2, dst, w, out, acc, srcbuf, dstbuf, wbuf, rows,
             sg0, sg1, ss0, ss1, si0, si1, si2, si3) = refs
        else:
            (dst, w, out, acc, dstbuf, wbuf, rows,
             ss0, ss1, si0, si1, si2, si3) = refs
            hs = src2 = srcbuf = None
            sg0 = sg1 = None
        sem_g = (sg0, sg1)
        sem_s = (ss0, ss1)
        sem_i = (si0, si1, si2, si3)
        c = lax.axis_index("c")
        s = lax.axis_index("s")

        zeros16 = jnp.zeros((LANES,), jnp.float32)

        def zrow(r, carry):
            for q in range(feat // LANES):
                rows[0, r, pl.ds(q * LANES, LANES)] = zeros16
            return carry

        lax.fori_loop(0, CHUNK, zrow, 0)
        for m in range(SLABS_PER_TILE):
            k = s + NS * m
            off = pl.multiple_of(k * CHUNK, CHUNK)

            @pl.when(k < N_FULL_SLABS)
            def _():
                pltpu.sync_copy(rows.at[0], acc.at[pl.ds(off, CHUNK)])

            @pl.when(k == N_FULL_SLABS)
            def _():
                pltpu.sync_copy(rows.at[0, pl.ds(0, TAIL_ROWS)],
                                acc.at[pl.ds(off, TAIL_ROWS)])

        tile = c * NS + s if split_cores else s
        base_off = c * N_NODES

        def issue_idx(i, q):
            e0 = pl.multiple_of((tile * n_chunk + i) * CHUNK, CHUNK)
            if gather:
                pltpu.async_copy(src2.at[c, pl.ds(e0, CHUNK)], srcbuf.at[q],
                                 sem_i[q])
            pltpu.async_copy(dst.at[pl.ds(e0, CHUNK)], dstbuf.at[q], sem_i[q])
            pltpu.async_copy(w.at[pl.ds(e0, CHUNK)], wbuf.at[q], sem_i[q])

        def drain_idx(q):
            if gather:
                pltpu.make_async_copy(dst.at[pl.ds(0, CHUNK)], srcbuf.at[q],
                                      sem_i[q]).wait()
            pltpu.make_async_copy(dst.at[pl.ds(0, CHUNK)], dstbuf.at[q],
                                  sem_i[q]).wait()
            pltpu.make_async_copy(w.at[pl.ds(0, CHUNK)], wbuf.at[q],
                                  sem_i[q]).wait()

        def issue_gather(q, b):
            pltpu.async_copy(hs.at[srcbuf.at[q]], rows.at[b], sem_g[b])

        def drain_gather(b):
            pltpu.make_async_copy(hs.at[pl.ds(0, CHUNK)], rows.at[b],
                                  sem_g[b]).wait()

        def issue_scatter(b, q):
            pltpu.async_copy(rows.at[b], acc.at[dstbuf.at[q]], sem_s[b],
                             add=True)

        def drain_scatter(b):
            pltpu.make_async_copy(out.at[pl.ds(0, CHUNK)], rows.at[b],
                                  sem_s[b]).wait()

        def scale(b, q):
            def sg(g, carry2):
                wv = wbuf[q, pl.ds(g * LANES, LANES)]
                for l in range(LANES):
                    wj = wv[l]
                    j = g * LANES + l
                    for fq in range(feat // LANES):
                        sl = pl.ds(fq * LANES, LANES)
                        if gather:
                            rows[b, j, sl] = rows[b, j, sl] * wj
                        else:
                            rows[b, j, sl] = rows[b, j, sl] * 0.0 + wj
                return carry2

            lax.fori_loop(0, CHUNK // LANES, sg, 0)

        issue_idx(0, 0)
        issue_idx(1, 1)
        drain_idx(0)
        if gather:
            issue_gather(0, 0)
        plsc.subcore_barrier()

        def quad(t, carry):
            for k in range(4):
                i = 4 * t + k
                b = k % 2
                q = k
                bn = 1 - b
                qn1 = (k + 1) % 4
                qn2 = (k + 2) % 4

                @pl.when(i >= 1)
                def _():
                    drain_scatter(bn)

                @pl.when(i + 1 < n_chunk)
                def _():
                    drain_idx(qn1)
                    if gather:
                        issue_gather(qn1, bn)

                @pl.when(i + 2 < n_chunk)
                def _():
                    issue_idx(i + 2, qn2)

                if gather:
                    drain_gather(b)
                scale(b, q)
                issue_scatter(b, q)
            return carry

        lax.fori_loop(0, n_chunk // 4, quad, 0)
        drain_scatter(1)
        plsc.subcore_barrier()

        for m in range(SLABS_PER_TILE):
            k = s + NS * m
            off = pl.multiple_of(k * CHUNK, CHUNK)

            @pl.when(k < N_FULL_SLABS)
            def _():
                pltpu.sync_copy(acc.at[pl.ds(off, CHUNK)],
                                out.at[pl.ds(base_off + off, CHUNK)])

            @pl.when(k == N_FULL_SLABS)
            def _():
                pltpu.sync_copy(acc.at[pl.ds(off, TAIL_ROWS)],
                                out.at[pl.ds(base_off + off, TAIL_ROWS)])

    scratch = [pltpu.VMEM_SHARED((N_NODES, feat), jnp.float32)]
    if gather:
        scratch.append(pltpu.VMEM((4, CHUNK), jnp.int32))
    scratch += [
        pltpu.VMEM((4, CHUNK), jnp.int32),
        pltpu.VMEM((4, CHUNK), jnp.float32),
        pltpu.VMEM((2, CHUNK, feat), jnp.float32),
    ]
    if gather:
        scratch += [pltpu.SemaphoreType.DMA, pltpu.SemaphoreType.DMA]
    scratch += [pltpu.SemaphoreType.DMA] * 6
    return pl.kernel(
        body,
        out_type=jax.ShapeDtypeStruct((NC * N_NODES, feat), jnp.float32),
        mesh=mesh,
        scratch_types=scratch,
    )


_agg_deg = _make_agg(128, split_cores=True, gather=False)
_agg_128 = _make_agg(128, split_cores=False)
_agg_64p = _make_agg(128, split_cores=True)


def _leaky(v):
    return jnp.where(v >= 0, v, 0.01 * v)


def _stage_a_body(degp_ref, x_ref, xs_ref, dinv_ref):
    dp = degp_ref[...]
    deg = dp[:, 0:1] + dp[:, 1:2] + 1.0
    dinv = jnp.where(deg > 0, lax.rsqrt(deg), 0.0)
    dinv_ref[...] = dinv
    xp = x_ref[...] * dinv
    xs_ref[0] = xp[:, :128]
    xs_ref[1] = xp[:, 128:]


_stage_a = pl.pallas_call(
    _stage_a_body,
    grid=(GRID,),
    in_specs=[
        pl.BlockSpec((RB, 2), lambda i: (i, 0)),
        pl.BlockSpec((RB, 256), lambda i: (i, 0)),
    ],
    out_specs=[
        pl.BlockSpec((2, RB, 128), lambda i: (0, i, 0)),
        pl.BlockSpec((RB, 1), lambda i: (i, 0)),
    ],
    out_shape=[
        jax.ShapeDtypeStruct((2, N_NODES, 128), jnp.float32),
        jax.ShapeDtypeStruct((N_NODES, 1), jnp.float32),
    ],
)


def _stage_c_body(s1_ref, xs_ref, dinv_ref, w1_ref, b1_ref, w2_ref, o_ref):
    d = dinv_ref[...]
    u0 = (s1_ref[0] + xs_ref[0]) * d
    u1 = (s1_ref[1] + xs_ref[1]) * d
    h = jnp.dot(u0, w1_ref[:128, :], preferred_element_type=jnp.float32)
    h = h + jnp.dot(u1, w1_ref[128:, :], preferred_element_type=jnp.float32)
    h = _leaky(h + b1_ref[...])
    t2 = jnp.dot(h, w2_ref[...], preferred_element_type=jnp.float32) * d
    o_ref[0] = t2[:, :128]
    o_ref[1] = t2[:, 128:]


_stage_c = pl.pallas_call(
    _stage_c_body,
    grid=(GRID,),
    in_specs=[
        pl.BlockSpec((2, RB, 128), lambda i: (0, i, 0)),
        pl.BlockSpec((2, RB, 128), lambda i: (0, i, 0)),
        pl.BlockSpec((RB, 1), lambda i: (i, 0)),
        pl.BlockSpec((256, 512), lambda i: (0, 0)),
        pl.BlockSpec((1, 512), lambda i: (0, 0)),
        pl.BlockSpec((512, 256), lambda i: (0, 0)),
    ],
    out_specs=pl.BlockSpec((2, RB, 128), lambda i: (0, i, 0)),
    out_shape=jax.ShapeDtypeStruct((2, N_NODES, 128), jnp.float32),
)


def _stage_e_body(s2_ref, t2s_ref, dinv_ref, b2_ref, w3_ref, o_ref):
    d = dinv_ref[...]
    u0 = (s2_ref[0] + t2s_ref[0]) * d
    u1 = (s2_ref[1] + t2s_ref[1]) * d
    h2a = _leaky(u0 + b2_ref[:, :128])
    h2b = _leaky(u1 + b2_ref[:, 128:])
    t3 = jnp.dot(h2a, w3_ref[:128, :], preferred_element_type=jnp.float32)
    t3 = t3 + jnp.dot(h2b, w3_ref[128:, :], preferred_element_type=jnp.float32)
    t3 = t3 * d
    o_ref[...] = jnp.concatenate([t3, jnp.zeros((RB, 64), jnp.float32)], axis=1)


_stage_e = pl.pallas_call(
    _stage_e_body,
    grid=(GRID,),
    in_specs=[
        pl.BlockSpec((2, RB, 128), lambda i: (0, i, 0)),
        pl.BlockSpec((2, RB, 128), lambda i: (0, i, 0)),
        pl.BlockSpec((RB, 1), lambda i: (i, 0)),
        pl.BlockSpec((1, 256), lambda i: (0, 0)),
        pl.BlockSpec((256, 64), lambda i: (0, 0)),
    ],
    out_specs=pl.BlockSpec((RB, 128), lambda i: (i, 0)),
    out_shape=jax.ShapeDtypeStruct((N_NODES, 128), jnp.float32),
)


def _stage_g_body(s3_ref, t3s_ref, dinv_ref, b3_ref, wc_ref, bc_ref, o_ref):
    d = dinv_ref[...]
    u = (s3_ref[0, :, :64] + s3_ref[1, :, :64] + t3s_ref[:, :64]) * d
    h3 = _leaky(u + b3_ref[...])
    o = jnp.dot(h3, wc_ref[...], preferred_element_type=jnp.float32)
    o_ref[...] = o + bc_ref[...]


_stage_g = pl.pallas_call(
    _stage_g_body,
    grid=(GRID,),
    in_specs=[
        pl.BlockSpec((2, RB, 128), lambda i: (0, i, 0)),
        pl.BlockSpec((RB, 128), lambda i: (i, 0)),
        pl.BlockSpec((RB, 1), lambda i: (i, 0)),
        pl.BlockSpec((1, 64), lambda i: (0, 0)),
        pl.BlockSpec((64, 128), lambda i: (0, 0)),
        pl.BlockSpec((1, 128), lambda i: (0, 0)),
    ],
    out_specs=pl.BlockSpec((RB, 128), lambda i: (i, 0)),
    out_shape=jax.ShapeDtypeStruct((N_NODES, 128), jnp.float32),
)


def kernel(x, edge_index, edge_attr, W1, b1, W2, b2, W3, b3, Wc, bc):
    src = edge_index[0].astype(jnp.int32)
    dst = edge_index[1].astype(jnp.int32)
    w = edge_attr.astype(jnp.float32)
    pad = E_PAD - N_EDGES
    src = jnp.concatenate([src, jnp.zeros((pad,), jnp.int32)])
    dst = jnp.concatenate([dst, jnp.zeros((pad,), jnp.int32)])
    w = jnp.concatenate([w, jnp.zeros((pad,), jnp.float32)])

    src_split = jnp.stack([src, src + N_NODES])
    src_same = jnp.stack([src, src])

    degp = _agg_deg(dst, w)
    degp2 = jnp.stack([degp[:N_NODES, 0], degp[N_NODES:, 0]], axis=1)

    xs, dinv = _stage_a(degp2, x)
    s1 = _agg_128(xs.reshape(NC * N_NODES, 128), src_split, dst, w)
    t2s = _stage_c(s1.reshape(2, N_NODES, 128), xs, dinv, W1,
                   b1.reshape(1, 512), W2)
    s2 = _agg_128(t2s.reshape(NC * N_NODES, 128), src_split, dst, w)
    t3s = _stage_e(s2.reshape(2, N_NODES, 128), t2s, dinv,
                   b2.reshape(1, 256), W3)
    s3 = _agg_64p(t3s, src_same, dst, w)
    wcp = jnp.pad(Wc, ((0, 0), (0, 125)))
    bcp = jnp.pad(bc, (0, 125)).reshape(1, 128)
    outp = _stage_g(s3.reshape(2, N_NODES, 128), t3s, dinv,
                    b3.reshape(1, 64), wcp, bcp)
    return outp[:, :3]
